# baseline (speedup 1.0000x reference)
import functools

import jax
import jax.numpy as jnp
from jax import lax
from jax.experimental import pallas as pl
from jax.experimental.pallas import tpu as pltpu
from jax.experimental.pallas import tpu_sc as plsc

L = 16
NC = 2
NS = 16
NW = NC * NS
UNROLL = 8


def _lookup_body(vocab_size, n_per_w, w_hbm, x_hbm, out_hbm, tbl_sh, idx_v,
                 out_v, sem_t, sem_i):
  wid = lax.axis_index("s") * NC + lax.axis_index("c")
  sid = lax.axis_index("s")
  base = wid * n_per_w

  icopy = pltpu.async_copy(x_hbm.at[pl.ds(base, n_per_w)], idx_v, sem_i)

  @pl.when(sid == 0)
  def _():
    pltpu.async_copy(w_hbm, tbl_sh, sem_t)

  icopy.wait()

  @plsc.parallel_loop(0, n_per_w, L, unroll=UNROLL)
  def _(off):
    ids = idx_v[pl.ds(off, L)]
    idx_v[pl.ds(off, L)] = jnp.where(ids >= vocab_size, 0, ids)

  @pl.when(sid == 0)
  def _():
    pltpu.make_async_copy(w_hbm, tbl_sh, sem_t).wait()

  plsc.subcore_barrier()

  pltpu.async_copy(tbl_sh.at[idx_v], out_v, sem_i).wait()
  pltpu.sync_copy(out_v, out_hbm.at[pl.ds(base, n_per_w)])


def kernel(x, weight):
  b, f = x.shape
  n = b * f
  vocab_size = weight.shape[0]
  n_per_w = n // NW
  assert n % (NW * L * UNROLL) == 0

  w_flat = weight.reshape(-1)
  x_flat = x.T.reshape(-1)

  mesh = plsc.VectorSubcoreMesh(core_axis_name="c", subcore_axis_name="s")
  run = pl.kernel(
      functools.partial(_lookup_body, vocab_size, n_per_w),
      out_type=jax.ShapeDtypeStruct((n,), jnp.int32),
      mesh=mesh,
      compiler_params=pltpu.CompilerParams(
          needs_layout_passes=False,
          skip_device_barrier=True,
          disable_semaphore_checks=True,
          disable_bounds_checks=True,
      ),
      scratch_types=[
          pltpu.VMEM_SHARED((vocab_size,), jnp.int32),
          pltpu.VMEM((n_per_w,), jnp.int32),
          pltpu.VMEM((n_per_w,), jnp.int32),
          pltpu.SemaphoreType.DMA,
          pltpu.SemaphoreType.DMA,
      ],
  )
  out = run(w_flat, x_flat)
  return out.reshape(f, 1, b).transpose(2, 0, 1)


# speedup vs baseline: 1.0005x; 1.0005x over previous
import functools

import jax
import jax.numpy as jnp
from jax import lax
from jax.experimental import pallas as pl
from jax.experimental.pallas import tpu as pltpu
from jax.experimental.pallas import tpu_sc as plsc

L = 16
NC = 2
NS = 16
NW = NC * NS
UNROLL = 16


def _lookup_body(vocab_size, n_per_w, w_hbm, x_hbm, out_hbm, tbl_sh, idx_v,
                 out_v, sem_t, sem_i):
  wid = lax.axis_index("s") * NC + lax.axis_index("c")
  sid = lax.axis_index("s")
  base = wid * n_per_w

  icopy = pltpu.async_copy(x_hbm.at[pl.ds(base, n_per_w)], idx_v, sem_i)

  @pl.when(sid == 0)
  def _():
    pltpu.async_copy(w_hbm, tbl_sh, sem_t)

  icopy.wait()

  @plsc.parallel_loop(0, n_per_w, L, unroll=UNROLL)
  def _(off):
    ids = idx_v[pl.ds(off, L)]
    idx_v[pl.ds(off, L)] = jnp.where(ids >= vocab_size, 0, ids)

  @pl.when(sid == 0)
  def _():
    pltpu.make_async_copy(w_hbm, tbl_sh, sem_t).wait()

  plsc.subcore_barrier()

  pltpu.async_copy(tbl_sh.at[idx_v], out_v, sem_i).wait()
  pltpu.sync_copy(out_v, out_hbm.at[pl.ds(base, n_per_w)])


def kernel(x, weight):
  b, f = x.shape
  n = b * f
  vocab_size = weight.shape[0]
  n_per_w = n // NW
  assert n % (NW * L * UNROLL) == 0

  w_flat = weight.reshape(-1)
  x_flat = x.T.reshape(-1)

  mesh = plsc.VectorSubcoreMesh(core_axis_name="c", subcore_axis_name="s")
  run = pl.kernel(
      functools.partial(_lookup_body, vocab_size, n_per_w),
      out_type=jax.ShapeDtypeStruct((n,), jnp.int32),
      mesh=mesh,
      compiler_params=pltpu.CompilerParams(
          needs_layout_passes=False,
          skip_device_barrier=True,
          disable_semaphore_checks=True,
          disable_bounds_checks=True,
      ),
      scratch_types=[
          pltpu.VMEM_SHARED((vocab_size,), jnp.int32),
          pltpu.VMEM((n_per_w,), jnp.int32),
          pltpu.VMEM((n_per_w,), jnp.int32),
          pltpu.SemaphoreType.DMA,
          pltpu.SemaphoreType.DMA,
      ],
  )
  out = run(w_flat, x_flat)
  return out.reshape(f, 1, b).transpose(2, 0, 1)
